# baseline (speedup 1.0000x reference)
import functools

import jax
import jax.numpy as jnp
from jax import lax
from jax.experimental import pallas as pl
from jax.experimental.pallas import tpu as pltpu
from jax.experimental.pallas import tpu_sc as plsc

B = 16384
D = 64
NC = 2
NS = 16
NW = NC * NS
BPW = B // NW
CHUNK = 128
NCH = BPW // CHUNK


def _sc_gather(c, table):
    mesh = plsc.VectorSubcoreMesh(core_axis_name="c", subcore_axis_name="s")

    @functools.partial(
        pl.kernel,
        mesh=mesh,
        out_type=jax.ShapeDtypeStruct((NW, NCH, CHUNK, D), jnp.float32),
        compiler_params=pltpu.CompilerParams(use_tc_tiling_on_sc=False),
        scratch_types=[
            pltpu.VMEM((NCH, CHUNK), jnp.int32),
            pltpu.VMEM((NCH, CHUNK, D), jnp.float32),
            pltpu.SemaphoreType.DMA,
        ],
    )
    def k(c_hbm, table_hbm, out_hbm, idx_v, rows_v, sem):
        wid = lax.axis_index("s") * NC + lax.axis_index("c")
        base = wid * BPW
        for j in range(NCH):
            pltpu.sync_copy(c_hbm.at[pl.ds(base + j * CHUNK, CHUNK)], idx_v.at[j])
        copies = [
            pltpu.async_copy(table_hbm.at[idx_v.at[j]], rows_v.at[j], sem)
            for j in range(NCH)
        ]
        for cp in copies:
            cp.wait()
        pltpu.sync_copy(rows_v, out_hbm.at[wid])

    return k(c, table)


def kernel(c, table):
    out = _sc_gather(c, table)
    return out.reshape(B, D)
